# Initial kernel scaffold; baseline (speedup 1.0000x reference)
import jax
import jax.numpy as jnp
from jax.experimental import pallas as pl


def kernel(preds):
    raise NotImplementedError("write your pallas kernel here")


# speedup vs baseline: 32.6021x; 32.6021x over previous
import jax
import jax.numpy as jnp
from jax.experimental import pallas as pl

_F = 128
_N_PAD = 5120
_CHUNK = 1280
_K = _N_PAD // _CHUNK


def _bbox_att_kernel(conf_ref, x1_ref, y1_ref, x2_ref, y2_ref, out_ref):
    k = pl.program_id(1)

    @pl.when(k == 0)
    def _init():
        out_ref[...] = jnp.zeros_like(out_ref)

    c = conf_ref[0]
    fx1 = jnp.floor(x1_ref[0] * _F).astype(jnp.int32)
    fy1 = jnp.floor(y1_ref[0] * _F).astype(jnp.int32)
    fx2 = jnp.floor(x2_ref[0] * _F).astype(jnp.int32)
    fy2 = jnp.floor(y2_ref[0] * _F).astype(jnp.int32)

    rows = jax.lax.broadcasted_iota(jnp.int32, (_F, _CHUNK), 0)
    ry = jnp.where((rows >= fy1) & (rows < fy2), c, 0.0)
    rx = ((rows >= fx1) & (rows < fx2)).astype(jnp.float32)

    acc = jax.lax.dot_general(
        ry, rx, (((1,), (1,)), ((), ())),
        preferred_element_type=jnp.float32)
    out_ref[0] += acc

    @pl.when(k == _K - 1)
    def _finish():
        out_ref[...] = jax.nn.sigmoid(out_ref[...])


def kernel(preds):
    B, N, _ = preds.shape
    pad = _N_PAD - N
    p = jnp.pad(preds, ((0, 0), (0, pad), (0, 0)))
    conf = p[:, :, 0].reshape(B, 1, _N_PAD)
    x1 = p[:, :, 1].reshape(B, 1, _N_PAD)
    y1 = p[:, :, 2].reshape(B, 1, _N_PAD)
    x2 = p[:, :, 3].reshape(B, 1, _N_PAD)
    y2 = p[:, :, 4].reshape(B, 1, _N_PAD)

    in_spec = pl.BlockSpec((1, 1, _CHUNK), lambda b, k: (b, 0, k))
    out_spec = pl.BlockSpec((1, _F, _F), lambda b, k: (b, 0, 0))
    return pl.pallas_call(
        _bbox_att_kernel,
        grid=(B, _K),
        in_specs=[in_spec] * 5,
        out_specs=out_spec,
        out_shape=jax.ShapeDtypeStruct((B, _F, _F), jnp.float32),
    )(conf, x1, y1, x2, y2)
